# baseline (speedup 1.0000x reference)
import functools

import jax
import jax.numpy as jnp
from jax import lax
from jax.experimental import pallas as pl
from jax.experimental.pallas import tpu as pltpu
from jax.experimental.pallas import tpu_sc as plsc

B = 8
C = 192
HW = 224 * 224
NCHK = 8
CH = HW // NCHK
ROWS = B * C
NUM_WORKERS = 32
R_PER_W = ROWS // NUM_WORKERS
W_PER_B = C // R_PER_W


def _body(x_hbm, perm_hbm, out_hbm, perm_v, idx_v, buf_v, gsem, osem):
    cid = lax.axis_index("c")
    sid = lax.axis_index("s")
    wid = sid * 2 + cid

    pltpu.sync_copy(perm_hbm, perm_v)
    b = wid // W_PER_B
    c0 = (wid % W_PER_B) * R_PER_W
    base = wid * R_PER_W
    lane = lax.iota(jnp.int32, 16)
    sub = lane & 7

    for g in range(R_PER_W // 16):
        pv = perm_v[pl.ds(c0 + g * 16, 16)]
        for m in range(8):
            srcs = jnp.where(lane < 8, pv[2 * m], pv[2 * m + 1])
            idx_v[pl.ds((g * 8 + m) * 16, 16)] = (b * C + srcs) * NCHK + sub

    def gather(t, slot):
        return pltpu.async_copy(
            x_hbm.at[idx_v.at[pl.ds(t * NCHK, NCHK)]],
            buf_v.at[pl.ds(slot * NCHK, NCHK)],
            gsem,
        )

    def put(t, slot):
        return pltpu.async_copy(
            buf_v.at[pl.ds(slot * NCHK, NCHK)],
            out_hbm.at[pl.ds((base + t) * NCHK, NCHK)],
            osem,
        )

    gh = {0: gather(0, 0), 1: gather(1, 1)}
    ph = {}
    for t in range(R_PER_W):
        s = t & 1
        if t >= 2:
            ph[t - 2].wait()
            gh[t] = gather(t, s)
        gh[t].wait()
        ph[t] = put(t, s)
    ph[R_PER_W - 2].wait()
    ph[R_PER_W - 1].wait()


_mesh = plsc.VectorSubcoreMesh(core_axis_name="c", subcore_axis_name="s")

_sc_permute = functools.partial(
    pl.kernel,
    mesh=_mesh,
    out_type=jax.ShapeDtypeStruct((ROWS * NCHK, CH), jnp.float32),
    scratch_types=[
        pltpu.VMEM((C,), jnp.int32),
        pltpu.VMEM((R_PER_W * NCHK,), jnp.int32),
        pltpu.VMEM((2 * NCHK, CH), jnp.float32),
        pltpu.SemaphoreType.DMA,
        pltpu.SemaphoreType.DMA,
    ],
)(_body)


def kernel(x, permutation):
    xf = x.reshape(ROWS * NCHK, CH)
    perm = permutation.astype(jnp.int32)
    out = _sc_permute(xf, perm)
    return out.reshape(B, C, 224, 224)


# speedup vs baseline: 1.0676x; 1.0676x over previous
import functools

import jax
import jax.numpy as jnp
from jax import lax
from jax.experimental import pallas as pl
from jax.experimental.pallas import tpu as pltpu
from jax.experimental.pallas import tpu_sc as plsc

B = 8
C = 192
HW = 224 * 224
ROWS = B * C
NUM_WORKERS = 32
R_PER_W = ROWS // NUM_WORKERS
W_PER_B = C // R_PER_W


def _body(x_hbm, perm_hbm, out_hbm, perm_v, buf_v, gsem, osem):
    cid = lax.axis_index("c")
    sid = lax.axis_index("s")
    wid = sid * 2 + cid

    pltpu.sync_copy(perm_hbm, perm_v)
    b = wid // W_PER_B
    c0 = (wid % W_PER_B) * R_PER_W
    base = wid * R_PER_W

    pvs = [perm_v[pl.ds(c0 + g * 16, 16)] for g in range(R_PER_W // 16)]

    def src_row(t):
        return b * C + pvs[t // 16][t % 16]

    def gather(t, slot):
        return pltpu.async_copy(
            x_hbm.at[pl.ds(src_row(t), 1)],
            buf_v.at[pl.ds(slot, 1)],
            gsem,
        )

    def put(t, slot):
        return pltpu.async_copy(
            buf_v.at[pl.ds(slot, 1)],
            out_hbm.at[pl.ds(base + t, 1)],
            osem,
        )

    gh = {0: gather(0, 0), 1: gather(1, 1)}
    ph = {}
    for t in range(R_PER_W):
        s = t & 1
        if t >= 2:
            ph[t - 2].wait()
            gh[t] = gather(t, s)
        gh[t].wait()
        ph[t] = put(t, s)
    ph[R_PER_W - 2].wait()
    ph[R_PER_W - 1].wait()


_mesh = plsc.VectorSubcoreMesh(core_axis_name="c", subcore_axis_name="s")

_sc_permute = functools.partial(
    pl.kernel,
    mesh=_mesh,
    out_type=jax.ShapeDtypeStruct((ROWS, HW), jnp.float32),
    scratch_types=[
        pltpu.VMEM((C,), jnp.int32),
        pltpu.VMEM((2, HW), jnp.float32),
        pltpu.SemaphoreType.DMA,
        pltpu.SemaphoreType.DMA,
    ],
)(_body)


def kernel(x, permutation):
    xf = x.reshape(ROWS, HW)
    perm = permutation.astype(jnp.int32)
    out = _sc_permute(xf, perm)
    return out.reshape(B, C, 224, 224)
